# Initial kernel scaffold; baseline (speedup 1.0000x reference)
import jax
import jax.numpy as jnp
from jax.experimental import pallas as pl


def kernel(adj_rows, adj_cols, adj_vals, feature_dense, user_emb, item_emb1, item_emb2, alpha0, alpha1, W_gc_0, b_gc_0, W_bi_0, b_bi_0, W_gc_1, b_gc_1, W_bi_1, b_bi_1, users, pos_items, neg_items):
    raise NotImplementedError("write your pallas kernel here")


# speedup vs baseline: 1.2221x; 1.2221x over previous
import functools

import jax
import jax.numpy as jnp
from jax import lax
from jax.experimental import pallas as pl
from jax.experimental.pallas import tpu as pltpu
from jax.experimental.pallas import tpu_sc as plsc

N_USER = 25000
N_ITEM = 25000
N = N_USER + N_ITEM
E = 800000
D = 96
NPART = 6
PARTS_PER_CORE = 3
NC = 2
NS = 16
LANES = 16

EDGES_PER_TILE = E // NS
CHUNK = 400
NCHUNK = EDGES_PER_TILE // CHUNK
SUB = 5
SUBW = CHUNK // SUB
ROWS_PER_TILE = N // NS

_mesh = plsc.VectorSubcoreMesh(core_axis_name="c", subcore_axis_name="s")
_sc_params = pltpu.CompilerParams(
    use_tc_tiling_on_sc=False, needs_layout_passes=False)


@functools.partial(
    pl.kernel,
    out_type=jax.ShapeDtypeStruct((N, D), jnp.float32),
    mesh=_mesh,
    compiler_params=_sc_params,
    scratch_types=[
        pltpu.VMEM_SHARED((N, LANES), jnp.float32),
        pltpu.VMEM((ROWS_PER_TILE, LANES), jnp.float32),
        pltpu.VMEM((CHUNK, LANES), jnp.float32),
        pltpu.VMEM((SUB, SUBW), jnp.int32),
        pltpu.VMEM((SUB, SUBW), jnp.int32),
        pltpu.VMEM((CHUNK,), jnp.float32),
        pltpu.SemaphoreType.DMA,
    ],
)
def _spmm_sc(x_flat, rows_hbm, cols_hbm, vals_hbm, out_hbm,
             acc, zbuf, gbuf, gidx, ridx, vals_v, sem):
    c = lax.axis_index("c")
    s = lax.axis_index("s")

    def zfill(i, _):
        zbuf[i, :] = jnp.zeros((LANES,), jnp.float32)
        return 0
    lax.fori_loop(0, ROWS_PER_TILE, zfill, 0)

    for p_local in range(PARTS_PER_CORE):
        p = c * PARTS_PER_CORE + p_local

        pltpu.sync_copy(zbuf, acc.at[pl.ds(s * ROWS_PER_TILE, ROWS_PER_TILE)])
        plsc.subcore_barrier()

        def chunk_body(k, _):
            base = s * EDGES_PER_TILE + k * CHUNK
            pltpu.sync_copy(vals_hbm.at[pl.ds(base, CHUNK)], vals_v)
            for j in range(SUB):
                pltpu.sync_copy(rows_hbm.at[pl.ds(base + j * SUBW, SUBW)],
                                ridx.at[j])
                pltpu.sync_copy(cols_hbm.at[pl.ds(base + j * SUBW, SUBW)],
                                gidx.at[j])
            def gixform(i, _):
                for j in range(SUB):
                    v = gidx[j, pl.ds(i * LANES, LANES)]
                    gidx[j, pl.ds(i * LANES, LANES)] = v * NPART + p
                return 0
            lax.fori_loop(0, SUBW // LANES, gixform, 0)
            cps = [
                pltpu.async_copy(x_flat.at[gidx.at[j]],
                                 gbuf.at[pl.ds(j * SUBW, SUBW)], sem)
                for j in range(SUB)
            ]
            for cp in cps:
                cp.wait()
            def scale(e, _):
                bval = plsc.load_gather(vals_v, [jnp.full((LANES,), e, jnp.int32)])
                gbuf[e, :] = gbuf[e, :] * bval
                return 0
            lax.fori_loop(0, CHUNK, scale, 0)
            for j in range(SUB):
                pltpu.sync_copy(gbuf.at[pl.ds(j * SUBW, SUBW)],
                                acc.at[ridx.at[j]], add=True)
            return 0
        lax.fori_loop(0, NCHUNK, chunk_body, 0)
        plsc.subcore_barrier()

        pltpu.sync_copy(acc.at[pl.ds(s * ROWS_PER_TILE, ROWS_PER_TILE)],
                        out_hbm.at[pl.ds(s * ROWS_PER_TILE, ROWS_PER_TILE),
                                   pl.ds(p * LANES, LANES)])
        plsc.subcore_barrier()


def _spmm(x, rows, cols, vals):
    return _spmm_sc(x.reshape(N * NPART, LANES), rows, cols, vals)


BLK = 400
NBLK = N // BLK


def _layer_tc_body(alpha_ref, ego_ref, h1_ref, h2_ref, h3_ref,
                   wgc_ref, bgc_ref, wbi_ref, bbi_ref, act_ref, out_ref):
    a0 = alpha_ref[0, 0]
    a1 = alpha_ref[0, 1]
    a2 = alpha_ref[0, 2]
    m = jnp.maximum(jnp.maximum(a0, a1), a2)
    e0 = jnp.exp(a0 - m)
    e1 = jnp.exp(a1 - m)
    e2 = jnp.exp(a2 - m)
    tot = e0 + e1 + e2
    b0 = e0 / tot
    b1 = e1 / tot
    b2 = e2 / tot
    side = b0 * h1_ref[...] + b1 * h2_ref[...] + b2 * h3_ref[...]
    ego = ego_ref[...]
    sum_e = jnp.dot(side, wgc_ref[...], preferred_element_type=jnp.float32) + bgc_ref[...]
    bi = jnp.dot(ego * side, wbi_ref[...], preferred_element_type=jnp.float32) + bbi_ref[...]
    act = jnp.where(sum_e >= 0, sum_e, 0.2 * sum_e) + bi
    act_ref[...] = act
    nrm = jnp.sqrt(jnp.sum(act * act, axis=1, keepdims=True))
    out_ref[...] = act / jnp.maximum(nrm, 1e-12)


def _layer_tc(alpha, ego, h1, h2, h3, wgc, bgc, wbi, bbi):
    return pl.pallas_call(
        _layer_tc_body,
        grid=(NBLK,),
        in_specs=[
            pl.BlockSpec(memory_space=pltpu.SMEM),
            pl.BlockSpec((BLK, D), lambda i: (i, 0)),
            pl.BlockSpec((BLK, D), lambda i: (i, 0)),
            pl.BlockSpec((BLK, D), lambda i: (i, 0)),
            pl.BlockSpec((BLK, D), lambda i: (i, 0)),
            pl.BlockSpec((D, D), lambda i: (0, 0)),
            pl.BlockSpec((1, D), lambda i: (0, 0)),
            pl.BlockSpec((D, D), lambda i: (0, 0)),
            pl.BlockSpec((1, D), lambda i: (0, 0)),
        ],
        out_specs=[pl.BlockSpec((BLK, D), lambda i: (i, 0)),
                   pl.BlockSpec((BLK, D), lambda i: (i, 0))],
        out_shape=[jax.ShapeDtypeStruct((N, D), jnp.float32),
                   jax.ShapeDtypeStruct((N, D), jnp.float32)],
    )(alpha.reshape(1, 3), ego, h1, h2, h3, wgc, bgc, wbi, bbi)


B3 = 3072
B_PER_W = B3 // (NC * NS)


@functools.partial(
    pl.kernel,
    out_type=[jax.ShapeDtypeStruct((B3, D), jnp.float32)] * 3,
    mesh=_mesh,
    compiler_params=_sc_params,
    scratch_types=[
        pltpu.VMEM((B_PER_W,), jnp.int32),
        pltpu.VMEM((B_PER_W, D), jnp.float32),
        pltpu.VMEM((B_PER_W, D), jnp.float32),
        pltpu.VMEM((B_PER_W, D), jnp.float32),
        pltpu.SemaphoreType.DMA,
    ],
)
def _batch_gather_sc(idx_hbm, t0, t1, t2, o0, o1, o2,
                     idx_v, r0, r1, r2, sem):
    c = lax.axis_index("c")
    s = lax.axis_index("s")
    wid = s * NC + c
    base = wid * B_PER_W
    pltpu.sync_copy(idx_hbm.at[pl.ds(base, B_PER_W)], idx_v)
    cp0 = pltpu.async_copy(t0.at[idx_v], r0, sem)
    cp1 = pltpu.async_copy(t1.at[idx_v], r1, sem)
    cp2 = pltpu.async_copy(t2.at[idx_v], r2, sem)
    cp0.wait()
    cp1.wait()
    cp2.wait()
    pltpu.sync_copy(r0, o0.at[pl.ds(base, B_PER_W)])
    pltpu.sync_copy(r1, o1.at[pl.ds(base, B_PER_W)])
    pltpu.sync_copy(r2, o2.at[pl.ds(base, B_PER_W)])


@jax.jit
def kernel(adj_rows, adj_cols, adj_vals, feature_dense, user_emb, item_emb1,
           item_emb2, alpha0, alpha1, W_gc_0, b_gc_0, W_bi_0, b_bi_0,
           W_gc_1, b_gc_1, W_bi_1, b_bi_1, users, pos_items, neg_items):
    ego0 = jnp.concatenate(
        [jnp.concatenate([user_emb, item_emb1], axis=0),
         jnp.concatenate([feature_dense, item_emb2], axis=0)], axis=1)

    alphas = [alpha0, alpha1]
    weights = [(W_gc_0, b_gc_0, W_bi_0, b_bi_0),
               (W_gc_1, b_gc_1, W_bi_1, b_bi_1)]
    ego = ego0
    embs = [ego0]
    for k in range(2):
        h1 = _spmm(ego, adj_rows, adj_cols, adj_vals)
        h2 = _spmm(h1, adj_rows, adj_cols, adj_vals)
        h3 = _spmm(h2, adj_rows, adj_cols, adj_vals)
        wgc, bgc, wbi, bbi = weights[k]
        ego, norm = _layer_tc(alphas[k], ego, h1, h2, h3, wgc, bgc, wbi, bbi)
        embs.append(norm)

    idx3 = jnp.concatenate(
        [users, N_USER + pos_items, N_USER + neg_items], axis=0)
    o0, o1, o2 = _batch_gather_sc(idx3, embs[0], embs[1], embs[2])
    return jnp.concatenate([o0, o1, o2], axis=1)


# speedup vs baseline: 6.7670x; 5.5371x over previous
import functools

import jax
import jax.numpy as jnp
from jax import lax
from jax.experimental import pallas as pl
from jax.experimental.pallas import tpu as pltpu
from jax.experimental.pallas import tpu_sc as plsc

N_USER = 25000
N_ITEM = 25000
N = N_USER + N_ITEM
E = 800000
D = 96
NPART = 6
PARTS_PER_CORE = 3
NC = 2
NS = 16
LANES = 16

EDGES_PER_TILE = E // NS
SUBW = 80
SUB = 5
BLK_E = SUB * SUBW
BIG = 10000
BIGROWS = BIG // SUBW
NBIG = EDGES_PER_TILE // BIG
NBLK_BIG = BIG // BLK_E
ROWS80_PER_TILE = EDGES_PER_TILE // SUBW
ROWS_PER_TILE = N // NS
ZROWS = 625

_mesh = plsc.VectorSubcoreMesh(core_axis_name="c", subcore_axis_name="s")
_sc_params = pltpu.CompilerParams(
    use_tc_tiling_on_sc=False, needs_layout_passes=False)


@functools.partial(
    pl.kernel,
    out_type=jax.ShapeDtypeStruct((N, D), jnp.float32),
    mesh=_mesh,
    compiler_params=_sc_params,
    scratch_types=[
        pltpu.VMEM_SHARED((N, LANES), jnp.float32),
        pltpu.VMEM((ZROWS, LANES), jnp.float32),
        pltpu.VMEM((BLK_E, LANES), jnp.float32),
        pltpu.VMEM((BLK_E, LANES), jnp.float32),
        pltpu.VMEM((BIGROWS, SUBW), jnp.int32),
        pltpu.VMEM((BIGROWS, SUBW), jnp.int32),
        pltpu.VMEM((BIG,), jnp.float32),
        pltpu.SemaphoreType.DMA,
        pltpu.SemaphoreType.DMA,
    ],
)
def _spmm_sc(x_flat, rows2d, cols2d, vals_hbm, out_hbm,
             acc, zbuf, gA, gB, cidx, ridx, vbuf, semA, semB):
    c = lax.axis_index("c")
    s = lax.axis_index("s")

    def zfill(i, _):
        zbuf[i, :] = jnp.zeros((LANES,), jnp.float32)
        return 0
    lax.fori_loop(0, ZROWS, zfill, 0)

    def fire(g, sem, b):
        for j in range(SUB):
            pltpu.async_copy(x_flat.at[cidx.at[b * SUB + j]],
                             g.at[pl.ds(j * SUBW, SUBW)], sem)

    def waitblk(g, sem, b):
        for j in range(SUB):
            pltpu.make_async_copy(x_flat.at[cidx.at[b * SUB + j]],
                                  g.at[pl.ds(j * SUBW, SUBW)], sem).wait()

    def scale(g, b):
        @plsc.parallel_loop(0, BLK_E, step=1, unroll=8)
        def _(e):
            bval = plsc.load_gather(
                vbuf, [jnp.full((LANES,), b * BLK_E + e, jnp.int32)])
            g[e, :] = g[e, :] * bval

    def scat(g, b):
        for j in range(SUB):
            pltpu.sync_copy(g.at[pl.ds(j * SUBW, SUBW)],
                            acc.at[ridx.at[b * SUB + j]], add=True)

    for p_local in range(PARTS_PER_CORE):
        p = c * PARTS_PER_CORE + p_local

        for z in range(ROWS_PER_TILE // ZROWS):
            pltpu.sync_copy(
                zbuf, acc.at[pl.ds(s * ROWS_PER_TILE + z * ZROWS, ZROWS)])
        plsc.subcore_barrier()

        def bigchunk(k, _):
            vbase = s * EDGES_PER_TILE + k * BIG
            rbase = s * ROWS80_PER_TILE + k * BIGROWS
            pltpu.sync_copy(vals_hbm.at[pl.ds(vbase, BIG)], vbuf)
            pltpu.sync_copy(rows2d.at[pl.ds(rbase, BIGROWS)], ridx)
            pltpu.sync_copy(cols2d.at[pl.ds(rbase, BIGROWS)], cidx)

            def tf(i, _):
                for t in range(SUBW // LANES):
                    sl = pl.ds(t * LANES, LANES)
                    cidx[i, sl] = cidx[i, sl] * NPART + p
                return 0
            lax.fori_loop(0, BIGROWS, tf, 0)

            fire(gA, semA, 0)

            def pair(m, _):
                b0 = 2 * m
                b1 = 2 * m + 1
                fire(gB, semB, b1)
                waitblk(gA, semA, b0)
                scale(gA, b0)
                scat(gA, b0)
                fire(gA, semA, b1 + 1)
                waitblk(gB, semB, b1)
                scale(gB, b1)
                scat(gB, b1)
                return 0
            lax.fori_loop(0, (NBLK_BIG - 1) // 2, pair, 0)

            last = NBLK_BIG - 1
            waitblk(gA, semA, last)
            scale(gA, last)
            scat(gA, last)
            return 0
        lax.fori_loop(0, NBIG, bigchunk, 0)
        plsc.subcore_barrier()

        pltpu.sync_copy(acc.at[pl.ds(s * ROWS_PER_TILE, ROWS_PER_TILE)],
                        out_hbm.at[pl.ds(s * ROWS_PER_TILE, ROWS_PER_TILE),
                                   pl.ds(p * LANES, LANES)])
        plsc.subcore_barrier()


def _spmm(x, rows, cols, vals):
    return _spmm_sc(x.reshape(N * NPART, LANES),
                    rows.reshape(E // SUBW, SUBW),
                    cols.reshape(E // SUBW, SUBW), vals)


BLK = 400
NBLK = N // BLK


def _layer_tc_body(alpha_ref, ego_ref, h1_ref, h2_ref, h3_ref,
                   wgc_ref, bgc_ref, wbi_ref, bbi_ref, act_ref, out_ref):
    a0 = alpha_ref[0, 0]
    a1 = alpha_ref[0, 1]
    a2 = alpha_ref[0, 2]
    m = jnp.maximum(jnp.maximum(a0, a1), a2)
    e0 = jnp.exp(a0 - m)
    e1 = jnp.exp(a1 - m)
    e2 = jnp.exp(a2 - m)
    tot = e0 + e1 + e2
    b0 = e0 / tot
    b1 = e1 / tot
    b2 = e2 / tot
    side = b0 * h1_ref[...] + b1 * h2_ref[...] + b2 * h3_ref[...]
    ego = ego_ref[...]
    sum_e = jnp.dot(side, wgc_ref[...], preferred_element_type=jnp.float32) + bgc_ref[...]
    bi = jnp.dot(ego * side, wbi_ref[...], preferred_element_type=jnp.float32) + bbi_ref[...]
    act = jnp.where(sum_e >= 0, sum_e, 0.2 * sum_e) + bi
    act_ref[...] = act
    nrm = jnp.sqrt(jnp.sum(act * act, axis=1, keepdims=True))
    out_ref[...] = act / jnp.maximum(nrm, 1e-12)


def _layer_tc(alpha, ego, h1, h2, h3, wgc, bgc, wbi, bbi):
    return pl.pallas_call(
        _layer_tc_body,
        grid=(NBLK,),
        in_specs=[
            pl.BlockSpec(memory_space=pltpu.SMEM),
            pl.BlockSpec((BLK, D), lambda i: (i, 0)),
            pl.BlockSpec((BLK, D), lambda i: (i, 0)),
            pl.BlockSpec((BLK, D), lambda i: (i, 0)),
            pl.BlockSpec((BLK, D), lambda i: (i, 0)),
            pl.BlockSpec((D, D), lambda i: (0, 0)),
            pl.BlockSpec((1, D), lambda i: (0, 0)),
            pl.BlockSpec((D, D), lambda i: (0, 0)),
            pl.BlockSpec((1, D), lambda i: (0, 0)),
        ],
        out_specs=[pl.BlockSpec((BLK, D), lambda i: (i, 0)),
                   pl.BlockSpec((BLK, D), lambda i: (i, 0))],
        out_shape=[jax.ShapeDtypeStruct((N, D), jnp.float32),
                   jax.ShapeDtypeStruct((N, D), jnp.float32)],
    )(alpha.reshape(1, 3), ego, h1, h2, h3, wgc, bgc, wbi, bbi)


B3 = 3072
B_PER_W = B3 // (NC * NS)


@functools.partial(
    pl.kernel,
    out_type=[jax.ShapeDtypeStruct((B3, D), jnp.float32)] * 3,
    mesh=_mesh,
    compiler_params=_sc_params,
    scratch_types=[
        pltpu.VMEM((B_PER_W,), jnp.int32),
        pltpu.VMEM((B_PER_W, D), jnp.float32),
        pltpu.VMEM((B_PER_W, D), jnp.float32),
        pltpu.VMEM((B_PER_W, D), jnp.float32),
        pltpu.SemaphoreType.DMA,
    ],
)
def _batch_gather_sc(idx_hbm, t0, t1, t2, o0, o1, o2,
                     idx_v, r0, r1, r2, sem):
    c = lax.axis_index("c")
    s = lax.axis_index("s")
    wid = s * NC + c
    base = wid * B_PER_W
    pltpu.sync_copy(idx_hbm.at[pl.ds(base, B_PER_W)], idx_v)
    cp0 = pltpu.async_copy(t0.at[idx_v], r0, sem)
    cp1 = pltpu.async_copy(t1.at[idx_v], r1, sem)
    cp2 = pltpu.async_copy(t2.at[idx_v], r2, sem)
    cp0.wait()
    cp1.wait()
    cp2.wait()
    pltpu.sync_copy(r0, o0.at[pl.ds(base, B_PER_W)])
    pltpu.sync_copy(r1, o1.at[pl.ds(base, B_PER_W)])
    pltpu.sync_copy(r2, o2.at[pl.ds(base, B_PER_W)])


@jax.jit
def kernel(adj_rows, adj_cols, adj_vals, feature_dense, user_emb, item_emb1,
           item_emb2, alpha0, alpha1, W_gc_0, b_gc_0, W_bi_0, b_bi_0,
           W_gc_1, b_gc_1, W_bi_1, b_bi_1, users, pos_items, neg_items):
    ego0 = jnp.concatenate(
        [jnp.concatenate([user_emb, item_emb1], axis=0),
         jnp.concatenate([feature_dense, item_emb2], axis=0)], axis=1)

    alphas = [alpha0, alpha1]
    weights = [(W_gc_0, b_gc_0, W_bi_0, b_bi_0),
               (W_gc_1, b_gc_1, W_bi_1, b_bi_1)]
    ego = ego0
    embs = [ego0]
    for k in range(2):
        h1 = _spmm(ego, adj_rows, adj_cols, adj_vals)
        h2 = _spmm(h1, adj_rows, adj_cols, adj_vals)
        h3 = _spmm(h2, adj_rows, adj_cols, adj_vals)
        wgc, bgc, wbi, bbi = weights[k]
        ego, norm = _layer_tc(alphas[k], ego, h1, h2, h3, wgc, bgc, wbi, bbi)
        embs.append(norm)

    idx3 = jnp.concatenate(
        [users, N_USER + pos_items, N_USER + neg_items], axis=0)
    o0, o1, o2 = _batch_gather_sc(idx3, embs[0], embs[1], embs[2])
    return jnp.concatenate([o0, o1, o2], axis=1)
